# Initial kernel scaffold; baseline (speedup 1.0000x reference)
import jax
import jax.numpy as jnp
from jax.experimental import pallas as pl


def kernel(x, edge_index, W1, b1, W2, b2):
    raise NotImplementedError("write your pallas kernel here")


# speedup vs baseline: 11.9025x; 11.9025x over previous
import functools

import jax
import jax.numpy as jnp
from jax import lax
from jax.experimental import pallas as pl
from jax.experimental.pallas import tpu as pltpu
from jax.experimental.pallas import tpu_sc as plsc

_NC = 2
_NS = 16
_K = 80


@functools.lru_cache(maxsize=None)
def _make_agg(N, E, F):
    NW = _NC * _NS
    EP = E // NW
    assert EP * NW == E and EP % _K == 0
    NCH = EP // _K
    RPT = (N // _NS) // 8 * 8
    REM = N - RPT * _NS
    assert REM % 8 == 0
    mesh = plsc.VectorSubcoreMesh(core_axis_name="c", subcore_axis_name="s")

    cparams = pltpu.CompilerParams(use_tc_tiling_on_sc=(F % 128 == 0))

    @functools.partial(
        pl.kernel,
        out_type=jax.ShapeDtypeStruct((_NC, N, F), jnp.float32),
        mesh=mesh,
        compiler_params=cparams,
        scratch_types=[
            pltpu.VMEM((_K, F), jnp.float32),
            pltpu.VMEM((_K,), jnp.int32),
            pltpu.VMEM((_K,), jnp.int32),
            pltpu.VMEM_SHARED((N, F), jnp.float32),
        ],
    )
    def agg(u_hbm, src_hbm, dst_hbm, z_hbm, out_hbm, gbuf, sidx, didx, acc):
        cid = lax.axis_index("c")
        sid = lax.axis_index("s")
        wid = sid * _NC + cid
        r0 = sid * RPT
        pltpu.sync_copy(z_hbm.at[pl.ds(r0, RPT)], acc.at[pl.ds(r0, RPT)])

        @pl.when(sid == _NS - 1)
        def _():
            pltpu.sync_copy(z_hbm.at[pl.ds(RPT * _NS, REM)],
                            acc.at[pl.ds(RPT * _NS, REM)])

        plsc.subcore_barrier()

        base = wid * EP

        @pl.loop(0, NCH)
        def _(ci):
            e0 = base + ci * _K
            pltpu.sync_copy(src_hbm.at[pl.ds(e0, _K)], sidx)
            pltpu.sync_copy(dst_hbm.at[pl.ds(e0, _K)], didx)
            pltpu.sync_copy(u_hbm.at[sidx], gbuf)
            pltpu.sync_copy(gbuf, acc.at[didx], add=True)

        plsc.subcore_barrier()
        pltpu.sync_copy(acc.at[pl.ds(r0, RPT)],
                        out_hbm.at[cid, pl.ds(r0, RPT)])

        @pl.when(sid == _NS - 1)
        def _():
            pltpu.sync_copy(acc.at[pl.ds(RPT * _NS, REM)],
                            out_hbm.at[cid, pl.ds(RPT * _NS, REM)])

    return agg


def _dinv_block(degp_ref):
    deg = degp_ref[0, :, 0] + degp_ref[1, :, 0] + 1.0
    return lax.rsqrt(deg)


def _tc1_body(degp_ref, x_ref, w_ref, u1_ref):
    dinv = _dinv_block(degp_ref)
    h = jnp.dot(x_ref[...], w_ref[...], preferred_element_type=jnp.float32)
    u1_ref[...] = h * dinv[:, None]


def _tc2_body(degp_ref, aggp_ref, u1_ref, b1_ref, w2_ref, u2_ref):
    dinv = _dinv_block(degp_ref)
    s = aggp_ref[0] + aggp_ref[1] + u1_ref[...]
    y1 = jnp.maximum(s * dinv[:, None] + b1_ref[...], 0.0)
    h2 = jnp.dot(y1, w2_ref[...], preferred_element_type=jnp.float32)
    u2_ref[...] = h2 * dinv[:, None]


def _tc3_body(degp_ref, aggp_ref, u2_ref, b2_ref, o_ref):
    dinv = _dinv_block(degp_ref)
    s = aggp_ref[0] + aggp_ref[1] + u2_ref[...]
    z = s * dinv[:, None] + b2_ref[...]
    o_ref[...] = 1.0 / (1.0 + jnp.exp(-z))


def kernel(x, edge_index, W1, b1, W2, b2):
    N, F = x.shape
    H = W1.shape[1]
    C = W2.shape[1]
    E = edge_index.shape[1]
    src = edge_index[0]
    dst = edge_index[1]

    BR = 1000
    grid = (N // BR,)

    ones_t = jnp.ones((N, 16), jnp.float32)
    z16 = jnp.zeros((N, 16), jnp.float32)
    zH = jnp.zeros((N, H), jnp.float32)

    agg16 = _make_agg(N, E, 16)
    aggH = _make_agg(N, E, H)

    degp = agg16(ones_t, src, dst, z16)

    u1 = pl.pallas_call(
        _tc1_body,
        grid=grid,
        in_specs=[
            pl.BlockSpec((2, BR, 16), lambda i: (0, i, 0)),
            pl.BlockSpec((BR, F), lambda i: (i, 0)),
            pl.BlockSpec((F, H), lambda i: (0, 0)),
        ],
        out_specs=pl.BlockSpec((BR, H), lambda i: (i, 0)),
        out_shape=jax.ShapeDtypeStruct((N, H), jnp.float32),
    )(degp, x, W1)

    agg1p = aggH(u1, src, dst, zH)

    u2 = pl.pallas_call(
        _tc2_body,
        grid=grid,
        in_specs=[
            pl.BlockSpec((2, BR, 16), lambda i: (0, i, 0)),
            pl.BlockSpec((2, BR, H), lambda i: (0, i, 0)),
            pl.BlockSpec((BR, H), lambda i: (i, 0)),
            pl.BlockSpec((1, H), lambda i: (0, 0)),
            pl.BlockSpec((H, C), lambda i: (0, 0)),
        ],
        out_specs=pl.BlockSpec((BR, C), lambda i: (i, 0)),
        out_shape=jax.ShapeDtypeStruct((N, C), jnp.float32),
    )(degp, agg1p, u1, b1.reshape(1, H), W2)

    agg2p = agg16(u2, src, dst, z16)

    out = pl.pallas_call(
        _tc3_body,
        grid=grid,
        in_specs=[
            pl.BlockSpec((2, BR, 16), lambda i: (0, i, 0)),
            pl.BlockSpec((2, BR, C), lambda i: (0, i, 0)),
            pl.BlockSpec((BR, C), lambda i: (i, 0)),
            pl.BlockSpec((1, C), lambda i: (0, 0)),
        ],
        out_specs=pl.BlockSpec((BR, C), lambda i: (i, 0)),
        out_shape=jax.ShapeDtypeStruct((N, C), jnp.float32),
    )(degp, agg2p, u2, b2.reshape(1, C))

    return out


# speedup vs baseline: 33.1752x; 2.7873x over previous
import functools

import jax
import jax.numpy as jnp
from jax import lax
from jax.experimental import pallas as pl
from jax.experimental.pallas import tpu as pltpu
from jax.experimental.pallas import tpu_sc as plsc

_NC = 2
_NS = 16
_K = 80


def _splits(N, E):
    NW = _NC * _NS
    EP = E // NW
    assert EP * NW == E and EP % _K == 0
    NCH = EP // _K
    RPT = (N // _NS) // 8 * 8
    REM = N - RPT * _NS
    assert REM % 8 == 0
    return EP, NCH, RPT, REM


def _zero_acc(z_hbm, acc, sid, RPT, REM):
    r0 = sid * RPT
    pltpu.sync_copy(z_hbm.at[pl.ds(r0, RPT)], acc.at[pl.ds(r0, RPT)])

    @pl.when(sid == _NS - 1)
    def _():
        pltpu.sync_copy(z_hbm.at[pl.ds(RPT * _NS, REM)],
                        acc.at[pl.ds(RPT * _NS, REM)])


def _write_out(acc, out_hbm, cid, sid, RPT, REM):
    r0 = sid * RPT
    pltpu.sync_copy(acc.at[pl.ds(r0, RPT)], out_hbm.at[cid, pl.ds(r0, RPT)])

    @pl.when(sid == _NS - 1)
    def _():
        pltpu.sync_copy(acc.at[pl.ds(RPT * _NS, REM)],
                        out_hbm.at[cid, pl.ds(RPT * _NS, REM)])


@functools.lru_cache(maxsize=None)
def _make_agg(N, E, F):
    EP, NCH, RPT, REM = _splits(N, E)
    mesh = plsc.VectorSubcoreMesh(core_axis_name="c", subcore_axis_name="s")

    cparams = pltpu.CompilerParams(use_tc_tiling_on_sc=(F % 128 == 0))

    @functools.partial(
        pl.kernel,
        out_type=jax.ShapeDtypeStruct((_NC, N, F), jnp.float32),
        mesh=mesh,
        compiler_params=cparams,
        scratch_types=[
            pltpu.VMEM((_K, F), jnp.float32),
            pltpu.VMEM((_K, F), jnp.float32),
            pltpu.VMEM((EP,), jnp.int32),
            pltpu.VMEM((EP,), jnp.int32),
            pltpu.VMEM_SHARED((N, F), jnp.float32),
            pltpu.SemaphoreType.DMA,
            pltpu.SemaphoreType.DMA,
        ],
    )
    def agg(u_hbm, src_hbm, dst_hbm, z_hbm, out_hbm,
            g0, g1, sidx, didx, acc, sem0, sem1):
        cid = lax.axis_index("c")
        sid = lax.axis_index("s")
        wid = sid * _NC + cid
        base = wid * EP
        pltpu.sync_copy(src_hbm.at[pl.ds(base, EP)], sidx)
        pltpu.sync_copy(dst_hbm.at[pl.ds(base, EP)], didx)
        _zero_acc(z_hbm, acc, sid, RPT, REM)
        plsc.subcore_barrier()

        def gather(c, buf, sem):
            return pltpu.async_copy(
                u_hbm.at[sidx.at[pl.ds(c * _K, _K)]], buf, sem)

        def wait_gather(c, buf, sem):
            pltpu.make_async_copy(
                u_hbm.at[sidx.at[pl.ds(c * _K, _K)]], buf, sem).wait()

        def scatter(c, buf):
            pltpu.sync_copy(buf, acc.at[didx.at[pl.ds(c * _K, _K)]],
                            add=True)

        gather(0, g0, sem0)

        @pl.loop(0, NCH, step=2)
        def _(c):
            @pl.when(c + 1 < NCH)
            def _():
                gather(c + 1, g1, sem1)

            wait_gather(c, g0, sem0)
            scatter(c, g0)

            @pl.when(c + 2 < NCH)
            def _():
                gather(c + 2, g0, sem0)

            @pl.when(c + 1 < NCH)
            def _():
                wait_gather(c + 1, g1, sem1)
                scatter(c + 1, g1)

        plsc.subcore_barrier()
        _write_out(acc, out_hbm, cid, sid, RPT, REM)

    return agg


@functools.lru_cache(maxsize=None)
def _make_deg(N, E):
    EP, NCH, RPT, REM = _splits(N, E)
    mesh = plsc.VectorSubcoreMesh(core_axis_name="c", subcore_axis_name="s")
    cparams = pltpu.CompilerParams(use_tc_tiling_on_sc=False)

    @functools.partial(
        pl.kernel,
        out_type=jax.ShapeDtypeStruct((_NC, N, 16), jnp.float32),
        mesh=mesh,
        compiler_params=cparams,
        scratch_types=[
            pltpu.VMEM((_K, 16), jnp.float32),
            pltpu.VMEM((EP,), jnp.int32),
            pltpu.VMEM_SHARED((N, 16), jnp.float32),
        ],
    )
    def deg(ones_hbm, dst_hbm, z_hbm, out_hbm, ones_v, didx, acc):
        cid = lax.axis_index("c")
        sid = lax.axis_index("s")
        wid = sid * _NC + cid
        pltpu.sync_copy(ones_hbm, ones_v)
        pltpu.sync_copy(dst_hbm.at[pl.ds(wid * EP, EP)], didx)
        _zero_acc(z_hbm, acc, sid, RPT, REM)
        plsc.subcore_barrier()

        @pl.loop(0, NCH)
        def _(c):
            pltpu.sync_copy(ones_v, acc.at[didx.at[pl.ds(c * _K, _K)]],
                            add=True)

        plsc.subcore_barrier()
        _write_out(acc, out_hbm, cid, sid, RPT, REM)

    return deg


def _dinv_block(degp_ref):
    deg = degp_ref[0, :, 0] + degp_ref[1, :, 0] + 1.0
    return lax.rsqrt(deg)


def _tc1_body(degp_ref, x_ref, w_ref, u1_ref):
    dinv = _dinv_block(degp_ref)
    h = jnp.dot(x_ref[...], w_ref[...], preferred_element_type=jnp.float32)
    u1_ref[...] = h * dinv[:, None]


def _tc2_body(degp_ref, aggp_ref, u1_ref, b1_ref, w2_ref, u2_ref):
    dinv = _dinv_block(degp_ref)
    s = aggp_ref[0] + aggp_ref[1] + u1_ref[...]
    y1 = jnp.maximum(s * dinv[:, None] + b1_ref[...], 0.0)
    h2 = jnp.dot(y1, w2_ref[...], preferred_element_type=jnp.float32)
    u2_ref[...] = h2 * dinv[:, None]


def _tc3_body(degp_ref, aggp_ref, u2_ref, b2_ref, o_ref):
    dinv = _dinv_block(degp_ref)
    s = aggp_ref[0] + aggp_ref[1] + u2_ref[...]
    z = s * dinv[:, None] + b2_ref[...]
    o_ref[...] = 1.0 / (1.0 + jnp.exp(-z))


def kernel(x, edge_index, W1, b1, W2, b2):
    N, F = x.shape
    H = W1.shape[1]
    C = W2.shape[1]
    E = edge_index.shape[1]
    src = edge_index[0]
    dst = edge_index[1]

    BR = 1000
    grid = (N // BR,)

    ones_k = jnp.ones((_K, 16), jnp.float32)
    z16 = jnp.zeros((N, 16), jnp.float32)
    zH = jnp.zeros((N, H), jnp.float32)

    agg16 = _make_agg(N, E, 16)
    aggH = _make_agg(N, E, H)

    degp = _make_deg(N, E)(ones_k, dst, z16)

    u1 = pl.pallas_call(
        _tc1_body,
        grid=grid,
        in_specs=[
            pl.BlockSpec((2, BR, 16), lambda i: (0, i, 0)),
            pl.BlockSpec((BR, F), lambda i: (i, 0)),
            pl.BlockSpec((F, H), lambda i: (0, 0)),
        ],
        out_specs=pl.BlockSpec((BR, H), lambda i: (i, 0)),
        out_shape=jax.ShapeDtypeStruct((N, H), jnp.float32),
    )(degp, x, W1)

    agg1p = aggH(u1, src, dst, zH)

    u2 = pl.pallas_call(
        _tc2_body,
        grid=grid,
        in_specs=[
            pl.BlockSpec((2, BR, 16), lambda i: (0, i, 0)),
            pl.BlockSpec((2, BR, H), lambda i: (0, i, 0)),
            pl.BlockSpec((BR, H), lambda i: (i, 0)),
            pl.BlockSpec((1, H), lambda i: (0, 0)),
            pl.BlockSpec((H, C), lambda i: (0, 0)),
        ],
        out_specs=pl.BlockSpec((BR, C), lambda i: (i, 0)),
        out_shape=jax.ShapeDtypeStruct((N, C), jnp.float32),
    )(degp, agg1p, u1, b1.reshape(1, H), W2)

    agg2p = agg16(u2, src, dst, z16)

    out = pl.pallas_call(
        _tc3_body,
        grid=grid,
        in_specs=[
            pl.BlockSpec((2, BR, 16), lambda i: (0, i, 0)),
            pl.BlockSpec((2, BR, C), lambda i: (0, i, 0)),
            pl.BlockSpec((BR, C), lambda i: (i, 0)),
            pl.BlockSpec((1, C), lambda i: (0, 0)),
        ],
        out_specs=pl.BlockSpec((BR, C), lambda i: (i, 0)),
        out_shape=jax.ShapeDtypeStruct((N, C), jnp.float32),
    )(degp, agg2p, u2, b2.reshape(1, C))

    return out


# speedup vs baseline: 37.9383x; 1.1436x over previous
import functools

import jax
import jax.numpy as jnp
from jax import lax
from jax.experimental import pallas as pl
from jax.experimental.pallas import tpu as pltpu
from jax.experimental.pallas import tpu_sc as plsc

_NC = 2
_NS = 16
_K = 80


def _splits(N, E, K):
    NW = _NC * _NS
    EP = E // NW
    assert EP * NW == E and EP % K == 0
    NCH = EP // K
    RPT = (N // _NS) // 8 * 8
    REM = N - RPT * _NS
    assert REM % 8 == 0
    return EP, NCH, RPT, REM


def _zero_acc(z_hbm, acc, sid, RPT, REM):
    r0 = sid * RPT
    pltpu.sync_copy(z_hbm.at[pl.ds(r0, RPT)], acc.at[pl.ds(r0, RPT)])

    @pl.when(sid == _NS - 1)
    def _():
        pltpu.sync_copy(z_hbm.at[pl.ds(RPT * _NS, REM)],
                        acc.at[pl.ds(RPT * _NS, REM)])


def _write_out(acc, out_hbm, cid, sid, RPT, REM):
    r0 = sid * RPT
    pltpu.sync_copy(acc.at[pl.ds(r0, RPT)], out_hbm.at[cid, pl.ds(r0, RPT)])

    @pl.when(sid == _NS - 1)
    def _():
        pltpu.sync_copy(acc.at[pl.ds(RPT * _NS, REM)],
                        out_hbm.at[cid, pl.ds(RPT * _NS, REM)])


@functools.lru_cache(maxsize=None)
def _make_agg(N, E, F):
    K, NB = (40, 5) if F >= 128 else (80, 6)
    EP, NCH, RPT, REM = _splits(N, E, K)
    assert NCH >= NB
    mesh = plsc.VectorSubcoreMesh(core_axis_name="c", subcore_axis_name="s")

    cparams = pltpu.CompilerParams(use_tc_tiling_on_sc=(F % 128 == 0))

    @functools.partial(
        pl.kernel,
        out_type=jax.ShapeDtypeStruct((_NC, N, F), jnp.float32),
        mesh=mesh,
        compiler_params=cparams,
        scratch_types=(
            [pltpu.VMEM((K, F), jnp.float32) for _ in range(NB)]
            + [
                pltpu.VMEM((EP,), jnp.int32),
                pltpu.VMEM((EP,), jnp.int32),
                pltpu.VMEM_SHARED((N, F), jnp.float32),
            ]
            + [pltpu.SemaphoreType.DMA for _ in range(2 * NB)]
        ),
    )
    def agg(u_hbm, src_hbm, dst_hbm, z_hbm, out_hbm, *rest):
        g = rest[:NB]
        sidx, didx, acc = rest[NB], rest[NB + 1], rest[NB + 2]
        gsem = rest[NB + 3:NB + 3 + NB]
        ssem = rest[NB + 3 + NB:]
        cid = lax.axis_index("c")
        sid = lax.axis_index("s")
        wid = sid * _NC + cid
        base = wid * EP
        pltpu.sync_copy(src_hbm.at[pl.ds(base, EP)], sidx)
        pltpu.sync_copy(dst_hbm.at[pl.ds(base, EP)], didx)
        _zero_acc(z_hbm, acc, sid, RPT, REM)
        plsc.subcore_barrier()

        def gather_start(c, b):
            pltpu.async_copy(u_hbm.at[sidx.at[pl.ds(c * K, K)]],
                             g[b], gsem[b])

        def gather_wait(c, b):
            pltpu.make_async_copy(u_hbm.at[sidx.at[pl.ds(c * K, K)]],
                                  g[b], gsem[b]).wait()

        def scat_start(c, b):
            pltpu.async_copy(g[b], acc.at[didx.at[pl.ds(c * K, K)]],
                             ssem[b], add=True)

        def scat_wait(c, b):
            pltpu.make_async_copy(g[b], acc.at[didx.at[pl.ds(c * K, K)]],
                                  ssem[b]).wait()

        gather_start(0, 0)
        gather_start(1, 1)

        @pl.loop(0, (NCH + NB - 1) // NB * NB, step=NB)
        def _(c0):
            for i in range(NB):
                c = c0 + i

                @pl.when(c < NCH)
                def _():
                    b2 = (i + 2) % NB

                    @pl.when(c + 2 < NCH)
                    def _():
                        @pl.when(c + 2 >= NB)
                        def _():
                            scat_wait(c + 2 - NB, b2)

                        gather_start(c + 2, b2)

                    gather_wait(c, i)
                    scat_start(c, i)

        for c in range(NCH - NB, NCH):
            scat_wait(c, c % NB)

        plsc.subcore_barrier()
        _write_out(acc, out_hbm, cid, sid, RPT, REM)

    return agg


@functools.lru_cache(maxsize=None)
def _make_deg(N, E):
    K = _K
    EP, NCH, RPT, REM = _splits(N, E, K)
    mesh = plsc.VectorSubcoreMesh(core_axis_name="c", subcore_axis_name="s")
    cparams = pltpu.CompilerParams(use_tc_tiling_on_sc=False)

    @functools.partial(
        pl.kernel,
        out_type=jax.ShapeDtypeStruct((_NC, N, 16), jnp.float32),
        mesh=mesh,
        compiler_params=cparams,
        scratch_types=[
            pltpu.VMEM((K, 16), jnp.float32),
            pltpu.VMEM((EP,), jnp.int32),
            pltpu.VMEM_SHARED((N, 16), jnp.float32),
            pltpu.SemaphoreType.DMA,
            pltpu.SemaphoreType.DMA,
            pltpu.SemaphoreType.DMA,
            pltpu.SemaphoreType.DMA,
        ],
    )
    def deg(ones_hbm, dst_hbm, z_hbm, out_hbm, ones_v, didx, acc, *ssem):
        cid = lax.axis_index("c")
        sid = lax.axis_index("s")
        wid = sid * _NC + cid
        pltpu.sync_copy(ones_hbm, ones_v)
        pltpu.sync_copy(dst_hbm.at[pl.ds(wid * EP, EP)], didx)
        _zero_acc(z_hbm, acc, sid, RPT, REM)
        plsc.subcore_barrier()

        def scat_start(c, b):
            pltpu.async_copy(ones_v, acc.at[didx.at[pl.ds(c * K, K)]],
                             ssem[b], add=True)

        def scat_wait(c, b):
            pltpu.make_async_copy(ones_v, acc.at[didx.at[pl.ds(c * K, K)]],
                                  ssem[b]).wait()

        @pl.loop(0, (NCH + 3) // 4 * 4, step=4)
        def _(c0):
            for i in range(4):
                c = c0 + i

                @pl.when(c < NCH)
                def _():
                    @pl.when(c >= 4)
                    def _():
                        scat_wait(c - 4, i)

                    scat_start(c, i)

        for c in range(NCH - 4, NCH):
            scat_wait(c, c % 4)

        plsc.subcore_barrier()
        _write_out(acc, out_hbm, cid, sid, RPT, REM)

    return deg


def _dinv_block(degp_ref):
    deg = degp_ref[0, :, 0] + degp_ref[1, :, 0] + 1.0
    return lax.rsqrt(deg)


def _tc1_body(degp_ref, x_ref, w_ref, u1_ref):
    dinv = _dinv_block(degp_ref)
    h = jnp.dot(x_ref[...], w_ref[...], preferred_element_type=jnp.float32)
    u1_ref[...] = h * dinv[:, None]


def _tc2_body(degp_ref, aggp_ref, u1_ref, b1_ref, w2_ref, u2_ref):
    dinv = _dinv_block(degp_ref)
    s = aggp_ref[0] + aggp_ref[1] + u1_ref[...]
    y1 = jnp.maximum(s * dinv[:, None] + b1_ref[...], 0.0)
    h2 = jnp.dot(y1, w2_ref[...], preferred_element_type=jnp.float32)
    u2_ref[...] = h2 * dinv[:, None]


def _tc3_body(degp_ref, aggp_ref, u2_ref, b2_ref, o_ref):
    dinv = _dinv_block(degp_ref)
    s = aggp_ref[0] + aggp_ref[1] + u2_ref[...]
    z = s * dinv[:, None] + b2_ref[...]
    o_ref[...] = 1.0 / (1.0 + jnp.exp(-z))


def kernel(x, edge_index, W1, b1, W2, b2):
    N, F = x.shape
    H = W1.shape[1]
    C = W2.shape[1]
    E = edge_index.shape[1]
    src = edge_index[0]
    dst = edge_index[1]

    BR = 1000
    grid = (N // BR,)

    ones_k = jnp.ones((_K, 16), jnp.float32)
    z16 = jnp.zeros((N, 16), jnp.float32)
    zH = jnp.zeros((N, H), jnp.float32)

    agg16 = _make_agg(N, E, 16)
    aggH = _make_agg(N, E, H)

    degp = _make_deg(N, E)(ones_k, dst, z16)

    u1 = pl.pallas_call(
        _tc1_body,
        grid=grid,
        in_specs=[
            pl.BlockSpec((2, BR, 16), lambda i: (0, i, 0)),
            pl.BlockSpec((BR, F), lambda i: (i, 0)),
            pl.BlockSpec((F, H), lambda i: (0, 0)),
        ],
        out_specs=pl.BlockSpec((BR, H), lambda i: (i, 0)),
        out_shape=jax.ShapeDtypeStruct((N, H), jnp.float32),
    )(degp, x, W1)

    agg1p = aggH(u1, src, dst, zH)

    u2 = pl.pallas_call(
        _tc2_body,
        grid=grid,
        in_specs=[
            pl.BlockSpec((2, BR, 16), lambda i: (0, i, 0)),
            pl.BlockSpec((2, BR, H), lambda i: (0, i, 0)),
            pl.BlockSpec((BR, H), lambda i: (i, 0)),
            pl.BlockSpec((1, H), lambda i: (0, 0)),
            pl.BlockSpec((H, C), lambda i: (0, 0)),
        ],
        out_specs=pl.BlockSpec((BR, C), lambda i: (i, 0)),
        out_shape=jax.ShapeDtypeStruct((N, C), jnp.float32),
    )(degp, agg1p, u1, b1.reshape(1, H), W2)

    agg2p = agg16(u2, src, dst, z16)

    out = pl.pallas_call(
        _tc3_body,
        grid=grid,
        in_specs=[
            pl.BlockSpec((2, BR, 16), lambda i: (0, i, 0)),
            pl.BlockSpec((2, BR, C), lambda i: (0, i, 0)),
            pl.BlockSpec((BR, C), lambda i: (i, 0)),
            pl.BlockSpec((1, C), lambda i: (0, 0)),
        ],
        out_specs=pl.BlockSpec((BR, C), lambda i: (i, 0)),
        out_shape=jax.ShapeDtypeStruct((N, C), jnp.float32),
    )(degp, agg2p, u2, b2.reshape(1, C))

    return out


# speedup vs baseline: 41.4963x; 1.0938x over previous
import functools

import jax
import jax.numpy as jnp
from jax import lax
from jax.experimental import pallas as pl
from jax.experimental.pallas import tpu as pltpu
from jax.experimental.pallas import tpu_sc as plsc

_NC = 2
_NS = 16
_K = 80


def _splits(N, E, K):
    NW = _NC * _NS
    EP = E // NW
    assert EP * NW == E and EP % K == 0
    NCH = EP // K
    RPT = (N // _NS) // 8 * 8
    REM = N - RPT * _NS
    assert REM % 8 == 0
    return EP, NCH, RPT, REM


def _zero_acc(z_hbm, zbuf, acc, sid, K, RPT, REM):
    r0 = sid * RPT
    nfull = RPT // K
    rem = RPT - nfull * K
    pltpu.sync_copy(z_hbm, zbuf)

    @pl.loop(0, nfull)
    def _(j):
        pltpu.sync_copy(zbuf, acc.at[pl.ds(r0 + j * K, K)])

    if rem:
        pltpu.sync_copy(zbuf.at[pl.ds(0, rem)],
                        acc.at[pl.ds(r0 + nfull * K, rem)])

    if REM:
        @pl.when(sid == _NS - 1)
        def _():
            pltpu.sync_copy(zbuf.at[pl.ds(0, REM)],
                            acc.at[pl.ds(RPT * _NS, REM)])


def _write_out(acc, out_hbm, cid, sid, RPT, REM):
    r0 = sid * RPT
    pltpu.sync_copy(acc.at[pl.ds(r0, RPT)], out_hbm.at[cid, pl.ds(r0, RPT)])

    @pl.when(sid == _NS - 1)
    def _():
        pltpu.sync_copy(acc.at[pl.ds(RPT * _NS, REM)],
                        out_hbm.at[cid, pl.ds(RPT * _NS, REM)])


@functools.lru_cache(maxsize=None)
def _make_agg(N, E, F):
    K, NB = (40, 5) if F >= 128 else (400, 6)
    EP, NCH, RPT, REM = _splits(N, E, K)
    assert NCH >= NB
    mesh = plsc.VectorSubcoreMesh(core_axis_name="c", subcore_axis_name="s")

    cparams = pltpu.CompilerParams(use_tc_tiling_on_sc=(F % 128 == 0))

    @functools.partial(
        pl.kernel,
        out_type=jax.ShapeDtypeStruct((_NC, N, F), jnp.float32),
        mesh=mesh,
        compiler_params=cparams,
        scratch_types=(
            [pltpu.VMEM((K, F), jnp.float32) for _ in range(NB)]
            + [
                pltpu.VMEM((EP,), jnp.int32),
                pltpu.VMEM((EP,), jnp.int32),
                pltpu.VMEM_SHARED((N, F), jnp.float32),
            ]
            + [pltpu.SemaphoreType.DMA for _ in range(2 * NB)]
        ),
    )
    def agg(u_hbm, src_hbm, dst_hbm, z_hbm, out_hbm, *rest):
        g = rest[:NB]
        sidx, didx, acc = rest[NB], rest[NB + 1], rest[NB + 2]
        gsem = rest[NB + 3:NB + 3 + NB]
        ssem = rest[NB + 3 + NB:]
        cid = lax.axis_index("c")
        sid = lax.axis_index("s")
        wid = sid * _NC + cid
        base = wid * EP
        pltpu.sync_copy(src_hbm.at[pl.ds(base, EP)], sidx)
        pltpu.sync_copy(dst_hbm.at[pl.ds(base, EP)], didx)
        _zero_acc(z_hbm, g[0], acc, sid, K, RPT, REM)
        plsc.subcore_barrier()

        def gather_start(c, b):
            pltpu.async_copy(u_hbm.at[sidx.at[pl.ds(c * K, K)]],
                             g[b], gsem[b])

        def gather_wait(c, b):
            pltpu.make_async_copy(u_hbm.at[sidx.at[pl.ds(c * K, K)]],
                                  g[b], gsem[b]).wait()

        def scat_start(c, b):
            pltpu.async_copy(g[b], acc.at[didx.at[pl.ds(c * K, K)]],
                             ssem[b], add=True)

        def scat_wait(c, b):
            pltpu.make_async_copy(g[b], acc.at[didx.at[pl.ds(c * K, K)]],
                                  ssem[b]).wait()

        gather_start(0, 0)
        gather_start(1, 1)

        @pl.loop(0, (NCH + NB - 1) // NB * NB, step=NB)
        def _(c0):
            for i in range(NB):
                c = c0 + i

                @pl.when(c < NCH)
                def _():
                    b2 = (i + 2) % NB

                    @pl.when(c + 2 < NCH)
                    def _():
                        @pl.when(c + 2 >= NB)
                        def _():
                            scat_wait(c + 2 - NB, b2)

                        gather_start(c + 2, b2)

                    gather_wait(c, i)
                    scat_start(c, i)

        for c in range(NCH - NB, NCH):
            scat_wait(c, c % NB)

        plsc.subcore_barrier()
        _write_out(acc, out_hbm, cid, sid, RPT, REM)

    return agg


@functools.lru_cache(maxsize=None)
def _make_deg(N, E):
    K = 400
    EP, NCH, RPT, REM = _splits(N, E, K)
    mesh = plsc.VectorSubcoreMesh(core_axis_name="c", subcore_axis_name="s")
    cparams = pltpu.CompilerParams(use_tc_tiling_on_sc=False)

    @functools.partial(
        pl.kernel,
        out_type=jax.ShapeDtypeStruct((_NC, N, 16), jnp.float32),
        mesh=mesh,
        compiler_params=cparams,
        scratch_types=[
            pltpu.VMEM((K, 16), jnp.float32),
            pltpu.VMEM((EP,), jnp.int32),
            pltpu.VMEM_SHARED((N, 16), jnp.float32),
            pltpu.SemaphoreType.DMA,
            pltpu.SemaphoreType.DMA,
            pltpu.SemaphoreType.DMA,
            pltpu.SemaphoreType.DMA,
        ],
    )
    def deg(ones_hbm, dst_hbm, z_hbm, out_hbm, ones_v, didx, acc, *ssem):
        cid = lax.axis_index("c")
        sid = lax.axis_index("s")
        wid = sid * _NC + cid
        pltpu.sync_copy(dst_hbm.at[pl.ds(wid * EP, EP)], didx)
        _zero_acc(z_hbm, ones_v, acc, sid, K, RPT, REM)
        pltpu.sync_copy(ones_hbm, ones_v)
        plsc.subcore_barrier()

        def scat_start(c, b):
            pltpu.async_copy(ones_v, acc.at[didx.at[pl.ds(c * K, K)]],
                             ssem[b], add=True)

        def scat_wait(c, b):
            pltpu.make_async_copy(ones_v, acc.at[didx.at[pl.ds(c * K, K)]],
                                  ssem[b]).wait()

        @pl.loop(0, (NCH + 3) // 4 * 4, step=4)
        def _(c0):
            for i in range(4):
                c = c0 + i

                @pl.when(c < NCH)
                def _():
                    @pl.when(c >= 4)
                    def _():
                        scat_wait(c - 4, i)

                    scat_start(c, i)

        for c in range(NCH - 4, NCH):
            scat_wait(c, c % 4)

        plsc.subcore_barrier()
        _write_out(acc, out_hbm, cid, sid, RPT, REM)

    return deg


def _dinv_block(degp_ref):
    deg = degp_ref[0, :, 0:1] + degp_ref[1, :, 0:1] + 1.0
    return lax.rsqrt(deg)


def _tc1_body(degp_ref, x_ref, w_ref, u1_ref):
    dinv = _dinv_block(degp_ref)
    h = jnp.dot(x_ref[...], w_ref[...], preferred_element_type=jnp.float32)
    u1_ref[...] = h * dinv


def _tc2_body(degp_ref, aggp_ref, u1_ref, b1_ref, w2_ref, u2_ref):
    dinv = _dinv_block(degp_ref)
    s = aggp_ref[0] + aggp_ref[1] + u1_ref[...]
    y1 = jnp.maximum(s * dinv + b1_ref[...], 0.0)
    h2 = jnp.dot(y1, w2_ref[...], preferred_element_type=jnp.float32)
    u2_ref[...] = h2 * dinv


def _tc3_body(degp_ref, aggp_ref, u2_ref, b2_ref, o_ref):
    dinv = _dinv_block(degp_ref)
    s = aggp_ref[0] + aggp_ref[1] + u2_ref[...]
    z = s * dinv + b2_ref[...]
    o_ref[...] = 1.0 / (1.0 + jnp.exp(-z))


def kernel(x, edge_index, W1, b1, W2, b2):
    N, F = x.shape
    H = W1.shape[1]
    C = W2.shape[1]
    E = edge_index.shape[1]
    src = edge_index[0]
    dst = edge_index[1]

    BR = 2000
    grid = (N // BR,)

    ones_k = jnp.ones((400, 16), jnp.float32)
    z16 = jnp.zeros((400, 16), jnp.float32)
    zH = jnp.zeros((40, H), jnp.float32)

    agg16 = _make_agg(N, E, 16)
    aggH = _make_agg(N, E, H)

    degp = _make_deg(N, E)(ones_k, dst, z16)

    u1 = pl.pallas_call(
        _tc1_body,
        grid=grid,
        in_specs=[
            pl.BlockSpec((2, BR, 16), lambda i: (0, i, 0)),
            pl.BlockSpec((BR, F), lambda i: (i, 0)),
            pl.BlockSpec((F, H), lambda i: (0, 0)),
        ],
        out_specs=pl.BlockSpec((BR, H), lambda i: (i, 0)),
        out_shape=jax.ShapeDtypeStruct((N, H), jnp.float32),
    )(degp, x, W1)

    agg1p = aggH(u1, src, dst, zH)

    u2 = pl.pallas_call(
        _tc2_body,
        grid=grid,
        in_specs=[
            pl.BlockSpec((2, BR, 16), lambda i: (0, i, 0)),
            pl.BlockSpec((2, BR, H), lambda i: (0, i, 0)),
            pl.BlockSpec((BR, H), lambda i: (i, 0)),
            pl.BlockSpec((1, H), lambda i: (0, 0)),
            pl.BlockSpec((H, C), lambda i: (0, 0)),
        ],
        out_specs=pl.BlockSpec((BR, C), lambda i: (i, 0)),
        out_shape=jax.ShapeDtypeStruct((N, C), jnp.float32),
    )(degp, agg1p, u1, b1.reshape(1, H), W2)

    agg2p = agg16(u2, src, dst, z16)

    out = pl.pallas_call(
        _tc3_body,
        grid=grid,
        in_specs=[
            pl.BlockSpec((2, BR, 16), lambda i: (0, i, 0)),
            pl.BlockSpec((2, BR, C), lambda i: (0, i, 0)),
            pl.BlockSpec((BR, C), lambda i: (i, 0)),
            pl.BlockSpec((1, C), lambda i: (0, 0)),
        ],
        out_specs=pl.BlockSpec((BR, C), lambda i: (i, 0)),
        out_shape=jax.ShapeDtypeStruct((N, C), jnp.float32),
    )(degp, agg2p, u2, b2.reshape(1, C))

    return out


# speedup vs baseline: 46.8086x; 1.1280x over previous
import functools

import jax
import jax.numpy as jnp
from jax import lax
from jax.experimental import pallas as pl
from jax.experimental.pallas import tpu as pltpu
from jax.experimental.pallas import tpu_sc as plsc

_NC = 2
_NS = 16
_K = 80


def _splits(N, E, K):
    NW = _NC * _NS
    EP = E // NW
    assert EP * NW == E and EP % K == 0
    NCH = EP // K
    RPT = (N // _NS) // 8 * 8
    REM = N - RPT * _NS
    assert REM % 8 == 0
    return EP, NCH, RPT, REM


def _zero_acc(z_hbm, zbuf, acc, sid, K, RPT, REM):
    r0 = sid * RPT
    nfull = RPT // K
    rem = RPT - nfull * K
    pltpu.sync_copy(z_hbm, zbuf)

    @pl.loop(0, nfull)
    def _(j):
        pltpu.sync_copy(zbuf, acc.at[pl.ds(r0 + j * K, K)])

    if rem:
        pltpu.sync_copy(zbuf.at[pl.ds(0, rem)],
                        acc.at[pl.ds(r0 + nfull * K, rem)])

    if REM:
        @pl.when(sid == _NS - 1)
        def _():
            pltpu.sync_copy(zbuf.at[pl.ds(0, REM)],
                            acc.at[pl.ds(RPT * _NS, REM)])


def _write_out(acc, out_hbm, cid, sid, RPT, REM):
    r0 = sid * RPT
    pltpu.sync_copy(acc.at[pl.ds(r0, RPT)], out_hbm.at[cid, pl.ds(r0, RPT)])

    @pl.when(sid == _NS - 1)
    def _():
        pltpu.sync_copy(acc.at[pl.ds(RPT * _NS, REM)],
                        out_hbm.at[cid, pl.ds(RPT * _NS, REM)])


@functools.lru_cache(maxsize=None)
def _make_agg(N, E, F):
    K, NB, D = (40, 6, 3) if F >= 128 else (400, 6, 2)
    EP, NCH, RPT, REM = _splits(N, E, K)
    assert NCH >= NB
    mesh = plsc.VectorSubcoreMesh(core_axis_name="c", subcore_axis_name="s")

    cparams = pltpu.CompilerParams(use_tc_tiling_on_sc=(F % 128 == 0))

    @functools.partial(
        pl.kernel,
        out_type=jax.ShapeDtypeStruct((_NC, N, F), jnp.float32),
        mesh=mesh,
        compiler_params=cparams,
        scratch_types=(
            [pltpu.VMEM((K, F), jnp.float32) for _ in range(NB)]
            + [
                pltpu.VMEM((EP,), jnp.int32),
                pltpu.VMEM((EP,), jnp.int32),
                pltpu.VMEM_SHARED((N, F), jnp.float32),
            ]
            + [pltpu.SemaphoreType.DMA for _ in range(2 * NB)]
        ),
    )
    def agg(u_hbm, src_hbm, dst_hbm, z_hbm, out_hbm, *rest):
        g = rest[:NB]
        sidx, didx, acc = rest[NB], rest[NB + 1], rest[NB + 2]
        gsem = rest[NB + 3:NB + 3 + NB]
        ssem = rest[NB + 3 + NB:]
        cid = lax.axis_index("c")
        sid = lax.axis_index("s")
        wid = sid * _NC + cid
        base = wid * EP
        pltpu.sync_copy(src_hbm.at[pl.ds(base, EP)], sidx)
        pltpu.sync_copy(dst_hbm.at[pl.ds(base, EP)], didx)
        _zero_acc(z_hbm, g[0], acc, sid, K, RPT, REM)
        plsc.subcore_barrier()

        def gather_start(c, b):
            pltpu.async_copy(u_hbm.at[sidx.at[pl.ds(c * K, K)]],
                             g[b], gsem[b])

        def gather_wait(c, b):
            pltpu.make_async_copy(u_hbm.at[sidx.at[pl.ds(c * K, K)]],
                                  g[b], gsem[b]).wait()

        def scat_start(c, b):
            pltpu.async_copy(g[b], acc.at[didx.at[pl.ds(c * K, K)]],
                             ssem[b], add=True)

        def scat_wait(c, b):
            pltpu.make_async_copy(g[b], acc.at[didx.at[pl.ds(c * K, K)]],
                                  ssem[b]).wait()

        for b in range(D):
            gather_start(b, b)

        @pl.loop(0, (NCH + NB - 1) // NB * NB, step=NB)
        def _(c0):
            for i in range(NB):
                c = c0 + i

                @pl.when(c < NCH)
                def _():
                    b2 = (i + D) % NB

                    @pl.when(c + D < NCH)
                    def _():
                        @pl.when(c + D >= NB)
                        def _():
                            scat_wait(c + D - NB, b2)

                        gather_start(c + D, b2)

                    gather_wait(c, i)
                    scat_start(c, i)

        for c in range(NCH - NB, NCH):
            scat_wait(c, c % NB)

        plsc.subcore_barrier()
        _write_out(acc, out_hbm, cid, sid, RPT, REM)

    return agg


@functools.lru_cache(maxsize=None)
def _make_deg(N, E):
    K = 400
    EP, NCH, RPT, REM = _splits(N, E, K)
    mesh = plsc.VectorSubcoreMesh(core_axis_name="c", subcore_axis_name="s")
    cparams = pltpu.CompilerParams(use_tc_tiling_on_sc=False)

    @functools.partial(
        pl.kernel,
        out_type=jax.ShapeDtypeStruct((_NC, N, 16), jnp.float32),
        mesh=mesh,
        compiler_params=cparams,
        scratch_types=[
            pltpu.VMEM((K, 16), jnp.float32),
            pltpu.VMEM((EP,), jnp.int32),
            pltpu.VMEM_SHARED((N, 16), jnp.float32),
            pltpu.SemaphoreType.DMA,
            pltpu.SemaphoreType.DMA,
            pltpu.SemaphoreType.DMA,
            pltpu.SemaphoreType.DMA,
        ],
    )
    def deg(ones_hbm, dst_hbm, z_hbm, out_hbm, ones_v, didx, acc, *ssem):
        cid = lax.axis_index("c")
        sid = lax.axis_index("s")
        wid = sid * _NC + cid
        pltpu.sync_copy(dst_hbm.at[pl.ds(wid * EP, EP)], didx)
        _zero_acc(z_hbm, ones_v, acc, sid, K, RPT, REM)
        pltpu.sync_copy(ones_hbm, ones_v)
        plsc.subcore_barrier()

        def scat_start(c, b):
            pltpu.async_copy(ones_v, acc.at[didx.at[pl.ds(c * K, K)]],
                             ssem[b], add=True)

        def scat_wait(c, b):
            pltpu.make_async_copy(ones_v, acc.at[didx.at[pl.ds(c * K, K)]],
                                  ssem[b]).wait()

        @pl.loop(0, (NCH + 3) // 4 * 4, step=4)
        def _(c0):
            for i in range(4):
                c = c0 + i

                @pl.when(c < NCH)
                def _():
                    @pl.when(c >= 4)
                    def _():
                        scat_wait(c - 4, i)

                    scat_start(c, i)

        for c in range(NCH - 4, NCH):
            scat_wait(c, c % 4)

        plsc.subcore_barrier()
        _write_out(acc, out_hbm, cid, sid, RPT, REM)

    return deg


def _dinv_block(degp_ref):
    deg = degp_ref[0, :, 0:1] + degp_ref[1, :, 0:1] + 1.0
    return lax.rsqrt(deg)


def _tc0_body(ei_ref, src_ref, dst_ref):
    src_ref[...] = ei_ref[0]
    dst_ref[...] = ei_ref[1]


def _tc1_body(degp_ref, x_ref, w_ref, u1_ref):
    dinv = _dinv_block(degp_ref)
    h = jnp.dot(x_ref[...], w_ref[...], preferred_element_type=jnp.float32)
    u1_ref[...] = h * dinv


def _tc2_body(degp_ref, aggp_ref, u1_ref, b1_ref, w2_ref, u2_ref):
    dinv = _dinv_block(degp_ref)
    s = aggp_ref[0] + aggp_ref[1] + u1_ref[...]
    y1 = jnp.maximum(s * dinv + b1_ref[...], 0.0)
    h2 = jnp.dot(y1, w2_ref[...], preferred_element_type=jnp.float32)
    u2_ref[...] = h2 * dinv


def _tc3_body(degp_ref, aggp_ref, u2_ref, b2_ref, o_ref):
    dinv = _dinv_block(degp_ref)
    s = aggp_ref[0] + aggp_ref[1] + u2_ref[...]
    z = s * dinv + b2_ref[...]
    o_ref[...] = 1.0 / (1.0 + jnp.exp(-z))


def kernel(x, edge_index, W1, b1, W2, b2):
    N, F = x.shape
    H = W1.shape[1]
    C = W2.shape[1]
    E = edge_index.shape[1]

    src, dst = pl.pallas_call(
        _tc0_body,
        grid=(1,),
        in_specs=[pl.BlockSpec((2, E), lambda i: (0, 0))],
        out_specs=[pl.BlockSpec((E,), lambda i: (0,)),
                   pl.BlockSpec((E,), lambda i: (0,))],
        out_shape=[jax.ShapeDtypeStruct((E,), jnp.int32)] * 2,
    )(edge_index)

    BR = 2000
    grid = (N // BR,)

    ones_k = jnp.ones((400, 16), jnp.float32)
    z16 = jnp.zeros((400, 16), jnp.float32)
    zH = jnp.zeros((40, H), jnp.float32)

    agg16 = _make_agg(N, E, 16)
    aggH = _make_agg(N, E, H)

    degp = _make_deg(N, E)(ones_k, dst, z16)

    u1 = pl.pallas_call(
        _tc1_body,
        grid=grid,
        in_specs=[
            pl.BlockSpec((2, BR, 16), lambda i: (0, i, 0)),
            pl.BlockSpec((BR, F), lambda i: (i, 0)),
            pl.BlockSpec((F, H), lambda i: (0, 0)),
        ],
        out_specs=pl.BlockSpec((BR, H), lambda i: (i, 0)),
        out_shape=jax.ShapeDtypeStruct((N, H), jnp.float32),
    )(degp, x, W1)

    agg1p = aggH(u1, src, dst, zH)

    u2 = pl.pallas_call(
        _tc2_body,
        grid=grid,
        in_specs=[
            pl.BlockSpec((2, BR, 16), lambda i: (0, i, 0)),
            pl.BlockSpec((2, BR, H), lambda i: (0, i, 0)),
            pl.BlockSpec((BR, H), lambda i: (i, 0)),
            pl.BlockSpec((1, H), lambda i: (0, 0)),
            pl.BlockSpec((H, C), lambda i: (0, 0)),
        ],
        out_specs=pl.BlockSpec((BR, C), lambda i: (i, 0)),
        out_shape=jax.ShapeDtypeStruct((N, C), jnp.float32),
    )(degp, agg1p, u1, b1.reshape(1, H), W2)

    agg2p = agg16(u2, src, dst, z16)

    out = pl.pallas_call(
        _tc3_body,
        grid=grid,
        in_specs=[
            pl.BlockSpec((2, BR, 16), lambda i: (0, i, 0)),
            pl.BlockSpec((2, BR, C), lambda i: (0, i, 0)),
            pl.BlockSpec((BR, C), lambda i: (i, 0)),
            pl.BlockSpec((1, C), lambda i: (0, 0)),
        ],
        out_specs=pl.BlockSpec((BR, C), lambda i: (i, 0)),
        out_shape=jax.ShapeDtypeStruct((N, C), jnp.float32),
    )(degp, agg2p, u2, b2.reshape(1, C))

    return out


# speedup vs baseline: 47.5929x; 1.0168x over previous
import functools

import jax
import jax.numpy as jnp
from jax import lax
from jax.experimental import pallas as pl
from jax.experimental.pallas import tpu as pltpu
from jax.experimental.pallas import tpu_sc as plsc

_NC = 2
_NS = 16
_K = 80


def _splits(N, E, K):
    NW = _NC * _NS
    EP = E // NW
    assert EP * NW == E and EP % K == 0
    NCH = EP // K
    RPT = (N // _NS) // 8 * 8
    REM = N - RPT * _NS
    assert REM % 8 == 0
    return EP, NCH, RPT, REM


def _zero_acc(z_hbm, zbuf, acc, sid, K, RPT, REM):
    r0 = sid * RPT
    nfull = RPT // K
    rem = RPT - nfull * K
    pltpu.sync_copy(z_hbm, zbuf)

    @pl.loop(0, nfull)
    def _(j):
        pltpu.sync_copy(zbuf, acc.at[pl.ds(r0 + j * K, K)])

    if rem:
        pltpu.sync_copy(zbuf.at[pl.ds(0, rem)],
                        acc.at[pl.ds(r0 + nfull * K, rem)])

    if REM:
        @pl.when(sid == _NS - 1)
        def _():
            pltpu.sync_copy(zbuf.at[pl.ds(0, REM)],
                            acc.at[pl.ds(RPT * _NS, REM)])


def _write_out(acc, out_hbm, cid, sid, RPT, REM):
    r0 = sid * RPT
    pltpu.sync_copy(acc.at[pl.ds(r0, RPT)], out_hbm.at[cid, pl.ds(r0, RPT)])

    @pl.when(sid == _NS - 1)
    def _():
        pltpu.sync_copy(acc.at[pl.ds(RPT * _NS, REM)],
                        out_hbm.at[cid, pl.ds(RPT * _NS, REM)])


@functools.lru_cache(maxsize=None)
def _make_agg(N, E, F):
    K, NB, D = (40, 6, 4) if F >= 128 else (400, 6, 3)
    EP, NCH, RPT, REM = _splits(N, E, K)
    assert NCH >= NB
    mesh = plsc.VectorSubcoreMesh(core_axis_name="c", subcore_axis_name="s")

    cparams = pltpu.CompilerParams(use_tc_tiling_on_sc=(F % 128 == 0))

    @functools.partial(
        pl.kernel,
        out_type=jax.ShapeDtypeStruct((_NC, N, F), jnp.float32),
        mesh=mesh,
        compiler_params=cparams,
        scratch_types=(
            [pltpu.VMEM((K, F), jnp.float32) for _ in range(NB)]
            + [
                pltpu.VMEM((EP,), jnp.int32),
                pltpu.VMEM((EP,), jnp.int32),
                pltpu.VMEM_SHARED((N, F), jnp.float32),
            ]
            + [pltpu.SemaphoreType.DMA for _ in range(2 * NB)]
        ),
    )
    def agg(u_hbm, src_hbm, dst_hbm, z_hbm, out_hbm, *rest):
        g = rest[:NB]
        sidx, didx, acc = rest[NB], rest[NB + 1], rest[NB + 2]
        gsem = rest[NB + 3:NB + 3 + NB]
        ssem = rest[NB + 3 + NB:]
        cid = lax.axis_index("c")
        sid = lax.axis_index("s")
        wid = sid * _NC + cid
        base = wid * EP
        pltpu.sync_copy(src_hbm.at[pl.ds(base, EP)], sidx)
        pltpu.sync_copy(dst_hbm.at[pl.ds(base, EP)], didx)
        _zero_acc(z_hbm, g[0], acc, sid, K, RPT, REM)
        plsc.subcore_barrier()

        def gather_start(c, b):
            pltpu.async_copy(u_hbm.at[sidx.at[pl.ds(c * K, K)]],
                             g[b], gsem[b])

        def gather_wait(c, b):
            pltpu.make_async_copy(u_hbm.at[sidx.at[pl.ds(c * K, K)]],
                                  g[b], gsem[b]).wait()

        def scat_start(c, b):
            pltpu.async_copy(g[b], acc.at[didx.at[pl.ds(c * K, K)]],
                             ssem[b], add=True)

        def scat_wait(c, b):
            pltpu.make_async_copy(g[b], acc.at[didx.at[pl.ds(c * K, K)]],
                                  ssem[b]).wait()

        for b in range(D):
            gather_start(b, b)

        @pl.loop(0, (NCH + NB - 1) // NB * NB, step=NB)
        def _(c0):
            for i in range(NB):
                c = c0 + i

                @pl.when(c < NCH)
                def _():
                    b2 = (i + D) % NB

                    @pl.when(c + D < NCH)
                    def _():
                        @pl.when(c + D >= NB)
                        def _():
                            scat_wait(c + D - NB, b2)

                        gather_start(c + D, b2)

                    gather_wait(c, i)
                    scat_start(c, i)

        for c in range(NCH - NB, NCH):
            scat_wait(c, c % NB)

        plsc.subcore_barrier()
        _write_out(acc, out_hbm, cid, sid, RPT, REM)

    return agg


@functools.lru_cache(maxsize=None)
def _make_deg(N, E):
    K = 400
    EP, NCH, RPT, REM = _splits(N, E, K)
    mesh = plsc.VectorSubcoreMesh(core_axis_name="c", subcore_axis_name="s")
    cparams = pltpu.CompilerParams(use_tc_tiling_on_sc=False)

    @functools.partial(
        pl.kernel,
        out_type=jax.ShapeDtypeStruct((_NC, N, 16), jnp.float32),
        mesh=mesh,
        compiler_params=cparams,
        scratch_types=[
            pltpu.VMEM((K, 16), jnp.float32),
            pltpu.VMEM((EP,), jnp.int32),
            pltpu.VMEM_SHARED((N, 16), jnp.float32),
            pltpu.SemaphoreType.DMA,
            pltpu.SemaphoreType.DMA,
            pltpu.SemaphoreType.DMA,
            pltpu.SemaphoreType.DMA,
        ],
    )
    def deg(ones_hbm, dst_hbm, z_hbm, out_hbm, ones_v, didx, acc, *ssem):
        cid = lax.axis_index("c")
        sid = lax.axis_index("s")
        wid = sid * _NC + cid
        pltpu.sync_copy(dst_hbm.at[pl.ds(wid * EP, EP)], didx)
        _zero_acc(z_hbm, ones_v, acc, sid, K, RPT, REM)
        pltpu.sync_copy(ones_hbm, ones_v)
        plsc.subcore_barrier()

        def scat_start(c, b):
            pltpu.async_copy(ones_v, acc.at[didx.at[pl.ds(c * K, K)]],
                             ssem[b], add=True)

        def scat_wait(c, b):
            pltpu.make_async_copy(ones_v, acc.at[didx.at[pl.ds(c * K, K)]],
                                  ssem[b]).wait()

        @pl.loop(0, (NCH + 3) // 4 * 4, step=4)
        def _(c0):
            for i in range(4):
                c = c0 + i

                @pl.when(c < NCH)
                def _():
                    @pl.when(c >= 4)
                    def _():
                        scat_wait(c - 4, i)

                    scat_start(c, i)

        for c in range(NCH - 4, NCH):
            scat_wait(c, c % 4)

        plsc.subcore_barrier()
        _write_out(acc, out_hbm, cid, sid, RPT, REM)

    return deg


def _dinv_block(degp_ref):
    deg = degp_ref[0, :, 0:1] + degp_ref[1, :, 0:1] + 1.0
    return lax.rsqrt(deg)


def _tc0_body(ei_ref, src_ref, dst_ref):
    src_ref[...] = ei_ref[0]
    dst_ref[...] = ei_ref[1]


def _tc1_body(degp_ref, x_ref, w_ref, u1_ref):
    dinv = _dinv_block(degp_ref)
    h = jnp.dot(x_ref[...], w_ref[...], preferred_element_type=jnp.float32)
    u1_ref[...] = h * dinv


def _tc2_body(degp_ref, aggp_ref, u1_ref, b1_ref, w2_ref, u2_ref):
    dinv = _dinv_block(degp_ref)
    s = aggp_ref[0] + aggp_ref[1] + u1_ref[...]
    y1 = jnp.maximum(s * dinv + b1_ref[...], 0.0)
    h2 = jnp.dot(y1, w2_ref[...], preferred_element_type=jnp.float32)
    u2_ref[...] = h2 * dinv


def _tc3_body(degp_ref, aggp_ref, u2_ref, b2_ref, o_ref):
    dinv = _dinv_block(degp_ref)
    s = aggp_ref[0] + aggp_ref[1] + u2_ref[...]
    z = s * dinv + b2_ref[...]
    o_ref[...] = 1.0 / (1.0 + jnp.exp(-z))


def kernel(x, edge_index, W1, b1, W2, b2):
    N, F = x.shape
    H = W1.shape[1]
    C = W2.shape[1]
    E = edge_index.shape[1]

    src, dst = pl.pallas_call(
        _tc0_body,
        grid=(1,),
        in_specs=[pl.BlockSpec((2, E), lambda i: (0, 0))],
        out_specs=[pl.BlockSpec((E,), lambda i: (0,)),
                   pl.BlockSpec((E,), lambda i: (0,))],
        out_shape=[jax.ShapeDtypeStruct((E,), jnp.int32)] * 2,
    )(edge_index)

    BR = 2000
    grid = (N // BR,)

    ones_k = jnp.ones((400, 16), jnp.float32)
    z16 = jnp.zeros((400, 16), jnp.float32)
    zH = jnp.zeros((40, H), jnp.float32)

    agg16 = _make_agg(N, E, 16)
    aggH = _make_agg(N, E, H)

    degp = _make_deg(N, E)(ones_k, dst, z16)

    u1 = pl.pallas_call(
        _tc1_body,
        grid=grid,
        in_specs=[
            pl.BlockSpec((2, BR, 16), lambda i: (0, i, 0)),
            pl.BlockSpec((BR, F), lambda i: (i, 0)),
            pl.BlockSpec((F, H), lambda i: (0, 0)),
        ],
        out_specs=pl.BlockSpec((BR, H), lambda i: (i, 0)),
        out_shape=jax.ShapeDtypeStruct((N, H), jnp.float32),
    )(degp, x, W1)

    agg1p = aggH(u1, src, dst, zH)

    u2 = pl.pallas_call(
        _tc2_body,
        grid=grid,
        in_specs=[
            pl.BlockSpec((2, BR, 16), lambda i: (0, i, 0)),
            pl.BlockSpec((2, BR, H), lambda i: (0, i, 0)),
            pl.BlockSpec((BR, H), lambda i: (i, 0)),
            pl.BlockSpec((1, H), lambda i: (0, 0)),
            pl.BlockSpec((H, C), lambda i: (0, 0)),
        ],
        out_specs=pl.BlockSpec((BR, C), lambda i: (i, 0)),
        out_shape=jax.ShapeDtypeStruct((N, C), jnp.float32),
    )(degp, agg1p, u1, b1.reshape(1, H), W2)

    agg2p = agg16(u2, src, dst, z16)

    out = pl.pallas_call(
        _tc3_body,
        grid=grid,
        in_specs=[
            pl.BlockSpec((2, BR, 16), lambda i: (0, i, 0)),
            pl.BlockSpec((2, BR, C), lambda i: (0, i, 0)),
            pl.BlockSpec((BR, C), lambda i: (i, 0)),
            pl.BlockSpec((1, C), lambda i: (0, 0)),
        ],
        out_specs=pl.BlockSpec((BR, C), lambda i: (i, 0)),
        out_shape=jax.ShapeDtypeStruct((N, C), jnp.float32),
    )(degp, agg2p, u2, b2.reshape(1, C))

    return out


# speedup vs baseline: 49.6982x; 1.0442x over previous
import functools

import jax
import jax.numpy as jnp
from jax import lax
from jax.experimental import pallas as pl
from jax.experimental.pallas import tpu as pltpu
from jax.experimental.pallas import tpu_sc as plsc

_NC = 2
_NS = 16
_K = 80


def _splits(N, E, K):
    NW = _NC * _NS
    EP = E // NW
    assert EP * NW == E and EP % K == 0
    NCH = EP // K
    RPT = (N // _NS) // 8 * 8
    REM = N - RPT * _NS
    assert REM % 8 == 0
    return EP, NCH, RPT, REM


def _zero_acc(z_hbm, zbuf, acc, sid, K, RPT, REM):
    r0 = sid * RPT
    nfull = RPT // K
    rem = RPT - nfull * K
    pltpu.sync_copy(z_hbm, zbuf)

    @pl.loop(0, nfull)
    def _(j):
        pltpu.sync_copy(zbuf, acc.at[pl.ds(r0 + j * K, K)])

    if rem:
        pltpu.sync_copy(zbuf.at[pl.ds(0, rem)],
                        acc.at[pl.ds(r0 + nfull * K, rem)])

    if REM:
        @pl.when(sid == _NS - 1)
        def _():
            pltpu.sync_copy(zbuf.at[pl.ds(0, REM)],
                            acc.at[pl.ds(RPT * _NS, REM)])


def _write_out(acc, out_hbm, cid, sid, RPT, REM):
    r0 = sid * RPT
    pltpu.sync_copy(acc.at[pl.ds(r0, RPT)], out_hbm.at[cid, pl.ds(r0, RPT)])

    @pl.when(sid == _NS - 1)
    def _():
        pltpu.sync_copy(acc.at[pl.ds(RPT * _NS, REM)],
                        out_hbm.at[cid, pl.ds(RPT * _NS, REM)])


@functools.lru_cache(maxsize=None)
def _make_agg(N, E, F):
    K, NB, D = (40, 6, 4) if F >= 128 else (400, 6, 3)
    EP, NCH, RPT, REM = _splits(N, E, K)
    assert NCH >= NB
    mesh = plsc.VectorSubcoreMesh(core_axis_name="c", subcore_axis_name="s")

    cparams = pltpu.CompilerParams(use_tc_tiling_on_sc=(F % 128 == 0))

    @functools.partial(
        pl.kernel,
        out_type=jax.ShapeDtypeStruct((_NC, N, F), jnp.float32),
        mesh=mesh,
        compiler_params=cparams,
        scratch_types=(
            [pltpu.VMEM((K, F), jnp.float32) for _ in range(NB)]
            + [
                pltpu.VMEM((EP,), jnp.int32),
                pltpu.VMEM((EP,), jnp.int32),
                pltpu.VMEM_SHARED((N, F), jnp.float32),
            ]
            + [pltpu.SemaphoreType.DMA for _ in range(2 * NB)]
        ),
    )
    def agg(u_hbm, src_hbm, dst_hbm, z_hbm, out_hbm, *rest):
        g = rest[:NB]
        sidx, didx, acc = rest[NB], rest[NB + 1], rest[NB + 2]
        gsem = rest[NB + 3:NB + 3 + NB]
        ssem = rest[NB + 3 + NB:]
        cid = lax.axis_index("c")
        sid = lax.axis_index("s")
        wid = sid * _NC + cid
        base = wid * EP
        pltpu.sync_copy(src_hbm.at[pl.ds(base, EP)], sidx)
        pltpu.sync_copy(dst_hbm.at[pl.ds(base, EP)], didx)
        _zero_acc(z_hbm, g[0], acc, sid, K, RPT, REM)
        plsc.subcore_barrier()

        def gather_start(c, b):
            pltpu.async_copy(u_hbm.at[sidx.at[pl.ds(c * K, K)]],
                             g[b], gsem[b])

        def gather_wait(c, b):
            pltpu.make_async_copy(u_hbm.at[sidx.at[pl.ds(c * K, K)]],
                                  g[b], gsem[b]).wait()

        def scat_start(c, b):
            pltpu.async_copy(g[b], acc.at[didx.at[pl.ds(c * K, K)]],
                             ssem[b], add=True)

        def scat_wait(c, b):
            pltpu.make_async_copy(g[b], acc.at[didx.at[pl.ds(c * K, K)]],
                                  ssem[b]).wait()

        for b in range(D):
            gather_start(b, b)

        @pl.loop(0, (NCH + NB - 1) // NB * NB, step=NB)
        def _(c0):
            for i in range(NB):
                c = c0 + i

                @pl.when(c < NCH)
                def _():
                    b2 = (i + D) % NB

                    @pl.when(c + D < NCH)
                    def _():
                        @pl.when(c + D >= NB)
                        def _():
                            scat_wait(c + D - NB, b2)

                        gather_start(c + D, b2)

                    gather_wait(c, i)
                    scat_start(c, i)

        for c in range(NCH - NB, NCH):
            scat_wait(c, c % NB)

        plsc.subcore_barrier()
        _write_out(acc, out_hbm, cid, sid, RPT, REM)

    return agg


@functools.lru_cache(maxsize=None)
def _make_deg(N, E):
    K = 400
    EP, NCH, RPT, REM = _splits(N, E, K)
    mesh = plsc.VectorSubcoreMesh(core_axis_name="c", subcore_axis_name="s")
    cparams = pltpu.CompilerParams(use_tc_tiling_on_sc=False)

    @functools.partial(
        pl.kernel,
        out_type=jax.ShapeDtypeStruct((_NC, N, 16), jnp.float32),
        mesh=mesh,
        compiler_params=cparams,
        scratch_types=[
            pltpu.VMEM((K, 16), jnp.float32),
            pltpu.VMEM((EP,), jnp.int32),
            pltpu.VMEM_SHARED((N, 16), jnp.float32),
            pltpu.SemaphoreType.DMA,
            pltpu.SemaphoreType.DMA,
            pltpu.SemaphoreType.DMA,
            pltpu.SemaphoreType.DMA,
        ],
    )
    def deg(ones_hbm, dst_hbm, z_hbm, out_hbm, ones_v, didx, acc, *ssem):
        cid = lax.axis_index("c")
        sid = lax.axis_index("s")
        wid = sid * _NC + cid
        pltpu.sync_copy(dst_hbm.at[pl.ds(wid * EP, EP)], didx)
        _zero_acc(z_hbm, ones_v, acc, sid, K, RPT, REM)
        pltpu.sync_copy(ones_hbm, ones_v)
        plsc.subcore_barrier()

        def scat_start(c, b):
            pltpu.async_copy(ones_v, acc.at[didx.at[pl.ds(c * K, K)]],
                             ssem[b], add=True)

        def scat_wait(c, b):
            pltpu.make_async_copy(ones_v, acc.at[didx.at[pl.ds(c * K, K)]],
                                  ssem[b]).wait()

        @pl.loop(0, (NCH + 3) // 4 * 4, step=4)
        def _(c0):
            for i in range(4):
                c = c0 + i

                @pl.when(c < NCH)
                def _():
                    @pl.when(c >= 4)
                    def _():
                        scat_wait(c - 4, i)

                    scat_start(c, i)

        for c in range(NCH - 4, NCH):
            scat_wait(c, c % 4)

        plsc.subcore_barrier()
        _write_out(acc, out_hbm, cid, sid, RPT, REM)

    return deg


def _tc0_body(ei_ref, src_ref, dst_ref):
    src_ref[...] = ei_ref[0]
    dst_ref[...] = ei_ref[1]


def _tc1_body(degp_ref, x_ref, w_ref, bc_ref, u1_ref, dw_ref):
    deg = degp_ref[0] + degp_ref[1] + 1.0
    dinv_w = jnp.dot(lax.rsqrt(deg), bc_ref[...],
                     preferred_element_type=jnp.float32)
    h = jnp.dot(x_ref[...], w_ref[...], preferred_element_type=jnp.float32)
    u1_ref[...] = h * dinv_w
    dw_ref[...] = dinv_w


def _tc2_body(dw_ref, aggp_ref, u1_ref, b1_ref, w2_ref, u2_ref):
    dinv_w = dw_ref[...]
    s = aggp_ref[0] + aggp_ref[1] + u1_ref[...]
    y1 = jnp.maximum(s * dinv_w + b1_ref[...], 0.0)
    h2 = jnp.dot(y1, w2_ref[...], preferred_element_type=jnp.float32)
    u2_ref[...] = h2 * dinv_w[:, 0:16]


def _tc3_body(degp_ref, aggp_ref, u2_ref, b2_ref, o_ref):
    dinv = lax.rsqrt(degp_ref[0] + degp_ref[1] + 1.0)
    s = aggp_ref[0] + aggp_ref[1] + u2_ref[...]
    z = s * dinv + b2_ref[...]
    o_ref[...] = 1.0 / (1.0 + jnp.exp(-z))


def kernel(x, edge_index, W1, b1, W2, b2):
    N, F = x.shape
    H = W1.shape[1]
    C = W2.shape[1]
    E = edge_index.shape[1]

    src, dst = pl.pallas_call(
        _tc0_body,
        grid=(1,),
        in_specs=[pl.BlockSpec((2, E), lambda i: (0, 0))],
        out_specs=[pl.BlockSpec((E,), lambda i: (0,)),
                   pl.BlockSpec((E,), lambda i: (0,))],
        out_shape=[jax.ShapeDtypeStruct((E,), jnp.int32)] * 2,
    )(edge_index)

    BR = 2000
    grid = (N // BR,)

    ones_k = jnp.ones((400, 16), jnp.float32)
    z16 = jnp.zeros((400, 16), jnp.float32)
    zH = jnp.zeros((40, H), jnp.float32)

    agg16 = _make_agg(N, E, 16)
    aggH = _make_agg(N, E, H)

    degp = _make_deg(N, E)(ones_k, dst, z16)

    bc = jnp.full((16, H), 1.0 / 16, jnp.float32)
    u1, dinv_w = pl.pallas_call(
        _tc1_body,
        grid=grid,
        in_specs=[
            pl.BlockSpec((2, BR, 16), lambda i: (0, i, 0)),
            pl.BlockSpec((BR, F), lambda i: (i, 0)),
            pl.BlockSpec((F, H), lambda i: (0, 0)),
            pl.BlockSpec((16, H), lambda i: (0, 0)),
        ],
        out_specs=[pl.BlockSpec((BR, H), lambda i: (i, 0)),
                   pl.BlockSpec((BR, H), lambda i: (i, 0))],
        out_shape=[jax.ShapeDtypeStruct((N, H), jnp.float32),
                   jax.ShapeDtypeStruct((N, H), jnp.float32)],
    )(degp, x, W1, bc)

    agg1p = aggH(u1, src, dst, zH)

    u2 = pl.pallas_call(
        _tc2_body,
        grid=grid,
        in_specs=[
            pl.BlockSpec((BR, H), lambda i: (i, 0)),
            pl.BlockSpec((2, BR, H), lambda i: (0, i, 0)),
            pl.BlockSpec((BR, H), lambda i: (i, 0)),
            pl.BlockSpec((1, H), lambda i: (0, 0)),
            pl.BlockSpec((H, C), lambda i: (0, 0)),
        ],
        out_specs=pl.BlockSpec((BR, C), lambda i: (i, 0)),
        out_shape=jax.ShapeDtypeStruct((N, C), jnp.float32),
    )(dinv_w, agg1p, u1, b1.reshape(1, H), W2)

    agg2p = agg16(u2, src, dst, z16)

    NP = N // 8
    BP = NP
    out_p = pl.pallas_call(
        _tc3_body,
        grid=(1,),
        in_specs=[
            pl.BlockSpec((2, BP, 128), lambda i: (0, i, 0)),
            pl.BlockSpec((2, BP, 128), lambda i: (0, i, 0)),
            pl.BlockSpec((BP, 128), lambda i: (i, 0)),
            pl.BlockSpec((1, 128), lambda i: (0, 0)),
        ],
        out_specs=pl.BlockSpec((BP, 128), lambda i: (i, 0)),
        out_shape=jax.ShapeDtypeStruct((NP, 128), jnp.float32),
    )(jnp.reshape(degp, (2, NP, 128)),
      jnp.reshape(agg2p, (2, NP, 128)),
      jnp.reshape(u2, (NP, 128)),
      jnp.tile(b2.reshape(1, C), (1, 128 // C)))

    return jnp.reshape(out_p, (N, C))
